# baseline (speedup 1.0000x reference)
import functools

import jax
import jax.numpy as jnp
from jax.experimental import pallas as pl
from jax.experimental.pallas import tpu as pltpu
from jax.experimental.pallas import tpu_sc as plsc

ALPHA = 0.5
T = 1.0


def _argmax2d(v, fiota):
    m = jnp.max(v)
    return jnp.min(jnp.where(v == m, fiota, 2147483647))


def _fps_kern(x_ref, y_ref, z_ref, *rest, npoint, weighted, out_rows):
    if weighted:
        w_ref, out_ref = rest
    else:
        (out_ref,) = rest
    x = x_ref[0]
    y = y_ref[0]
    z = z_ref[0]
    R, C = x.shape
    riota = jax.lax.broadcasted_iota(jnp.int32, (R, C), 0)
    ciota = jax.lax.broadcasted_iota(jnp.int32, (R, C), 1)
    fiota = riota * C + ciota
    oriota = jax.lax.broadcasted_iota(jnp.int32, (out_rows, 128), 0)
    ociota = jax.lax.broadcasted_iota(jnp.int32, (out_rows, 128), 1)
    oiota = oriota * 128 + ociota
    if weighted:
        w = w_ref[0]
        far0 = _argmax2d(w, fiota)
    else:
        far0 = jnp.int32(0)
    dists0 = jnp.full((R, C), 1e10, jnp.float32)
    inds0 = jnp.zeros((out_rows, 128), jnp.int32)

    def step(i, carry):
        dists, inds, far = carry
        sel = fiota == far
        ninf = jnp.float32(-jnp.inf)
        cx = jnp.max(jnp.where(sel, x, ninf))
        cy = jnp.max(jnp.where(sel, y, ninf))
        cz = jnp.max(jnp.where(sel, z, ninf))
        dx = x - cx
        dy = y - cy
        dz = z - cz
        d = dx * dx + dy * dy + dz * dz
        dists = jnp.minimum(dists, d)
        inds = jnp.where(oiota == i, far, inds)
        if weighted:
            score = jnp.maximum(dists, 0.0) * w
        else:
            score = dists
        nxt = _argmax2d(score, fiota)
        return dists, inds, nxt

    _, inds, _ = jax.lax.fori_loop(0, npoint, step, (dists0, inds0, far0))
    out_ref[0] = inds


def _fps_pallas(xyz, npoint, w=None):
    B, N, _ = xyz.shape
    C = 128
    R = N // C
    assert R * C == N
    out_rows = max(1, (npoint + 127) // 128)
    xr = xyz[:, :, 0].reshape(B, R, C)
    yr = xyz[:, :, 1].reshape(B, R, C)
    zr = xyz[:, :, 2].reshape(B, R, C)
    weighted = w is not None
    args = [xr, yr, zr]
    if weighted:
        args.append(w.reshape(B, R, C))
    spec = pl.BlockSpec((1, R, C), lambda b: (b, 0, 0))
    in_specs = [spec] * len(args)
    out = pl.pallas_call(
        functools.partial(_fps_kern, npoint=npoint, weighted=weighted,
                          out_rows=out_rows),
        grid=(B,),
        in_specs=in_specs,
        out_specs=pl.BlockSpec((1, out_rows, 128), lambda b: (b, 0, 0)),
        out_shape=jax.ShapeDtypeStruct((B, out_rows, 128), jnp.int32),
    )(*args)
    return out.reshape(B, out_rows * 128)[:, :npoint]


def _bq_sc_body(cx_h, cy_h, cz_h, x_h, y_h, z_h, out_h,
                xs, ys, zs, cdd, cdi, outv, cxs, cys, czs,
                *, B, S, N, K, KS, WPB, RPW, r2, NC):
    c = jax.lax.axis_index("c")
    s = jax.lax.axis_index("s")
    wid = s * NC + c
    b = wid // WPB
    ww = wid % WPB
    pltpu.sync_copy(x_h.at[b], xs)
    pltpu.sync_copy(y_h.at[b], ys)
    pltpu.sync_copy(z_h.at[b], zs)
    base = ww * RPW
    pltpu.sync_copy(cx_h.at[b, pl.ds(base, RPW)], cxs)
    pltpu.sync_copy(cy_h.at[b, pl.ds(base, RPW)], cys)
    pltpu.sync_copy(cz_h.at[b, pl.ds(base, RPW)], czs)
    iota = jax.lax.iota(jnp.int32, 16)
    NCH = N // 16

    def row_fn(r, _):
        c0 = (r // 16) * 16
        lsel = iota == (r - c0)
        lself = lsel.astype(jnp.float32)
        cxv = jnp.full((16,), jnp.sum(cxs[pl.ds(c0, 16)] * lself), jnp.float32)
        cyv = jnp.full((16,), jnp.sum(cys[pl.ds(c0, 16)] * lself), jnp.float32)
        czv = jnp.full((16,), jnp.sum(czs[pl.ds(c0, 16)] * lself), jnp.float32)

        def scan_ch(j, car):
            cnt, minv, mini = car
            sl = pl.ds(j * 16, 16)
            dx = xs[sl] - cxv
            dy = ys[sl] - cyv
            dz = zs[sl] - czv
            d2 = dx * dx + dy * dy + dz * dz
            d2c = jnp.maximum(d2, 0.0)
            iv = iota + j * 16
            m = d2 <= r2
            plsc.store_compressed(cdd.at[pl.ds(cnt, 16)], d2c, mask=m)
            plsc.store_compressed(cdi.at[pl.ds(cnt, 16)], iv, mask=m)
            cnt = cnt + jnp.sum(m.astype(jnp.int32))
            mm = d2c < minv
            minv = jnp.where(mm, d2c, minv)
            mini = jnp.where(mm, iv, mini)
            return cnt, minv, mini

        cnt, minv, mini = jax.lax.fori_loop(
            0, NCH, scan_ch,
            (jnp.int32(0), jnp.full((16,), jnp.inf, jnp.float32),
             jnp.zeros((16,), jnp.int32)))
        mn = jnp.min(minv)
        nearest = jnp.min(jnp.where(minv == mn, mini, 2147483647))
        nchc = (cnt + 15) // 16

        def bit_fn(i, acc):
            t = acc | (1 << (30 - i))

            def cnt_ch(j, c2):
                bits = plsc.bitcast(cdd[pl.ds(j * 16, 16)], jnp.int32)
                le = (bits < t) & ((iota + j * 16) < cnt)
                return c2 + jnp.sum(le.astype(jnp.int32))

            cless = jax.lax.fori_loop(0, nchc, cnt_ch, jnp.int32(0))
            return jnp.where(cless < K, t, acc)

        tau = jax.lax.fori_loop(0, 31, bit_fn, jnp.int32(0))
        nearv = jnp.full((16,), nearest, jnp.int32)
        for cslot in range(KS // 16):
            outv[cslot * 16:(cslot + 1) * 16] = nearv

        def emit_ch(j, pos):
            sl = pl.ds(j * 16, 16)
            bits = plsc.bitcast(cdd[sl], jnp.int32)
            le = (bits <= tau) & ((iota + j * 16) < cnt)
            pref = plsc.cumsum(le.astype(jnp.int32))
            keep = le & ((pref + pos) <= K)
            plsc.store_compressed(outv.at[pl.ds(pos, 16)], cdi[sl], mask=keep)
            return pos + jnp.sum(keep.astype(jnp.int32))

        jax.lax.fori_loop(0, nchc, emit_ch, jnp.int32(0))
        row = b * S + base + r
        pltpu.sync_copy(outv.at[pl.ds(0, K)], out_h.at[row])
        return 0

    jax.lax.fori_loop(0, RPW, row_fn, 0)


def _ball_query_sc(new_xyz, xyz, radius, nsample):
    B, S, _ = new_xyz.shape
    N = xyz.shape[1]
    info = plsc.get_sparse_core_info()
    NC, NS = info.num_cores, info.num_subcores
    NW = NC * NS
    WPB = NW // B
    RPW = S // WPB
    K = nsample
    KS = K + 16
    r2 = float(radius) * float(radius)
    body = functools.partial(_bq_sc_body, B=B, S=S, N=N, K=K, KS=KS,
                             WPB=WPB, RPW=RPW, r2=r2, NC=NC)
    mesh = plsc.VectorSubcoreMesh(core_axis_name="c", subcore_axis_name="s")
    fn = pl.kernel(
        body, mesh=mesh,
        out_type=jax.ShapeDtypeStruct((B * S, K), jnp.int32),
        scratch_types=[
            pltpu.VMEM((N,), jnp.float32),
            pltpu.VMEM((N,), jnp.float32),
            pltpu.VMEM((N,), jnp.float32),
            pltpu.VMEM((N,), jnp.float32),
            pltpu.VMEM((N,), jnp.int32),
            pltpu.VMEM((KS,), jnp.int32),
            pltpu.VMEM((RPW,), jnp.float32),
            pltpu.VMEM((RPW,), jnp.float32),
            pltpu.VMEM((RPW,), jnp.float32),
        ])
    out = fn(new_xyz[..., 0].reshape(B, S), new_xyz[..., 1].reshape(B, S),
             new_xyz[..., 2].reshape(B, S), xyz[..., 0].reshape(B, N),
             xyz[..., 1].reshape(B, N), xyz[..., 2].reshape(B, N))
    return out.reshape(B, S, K)


def _conv1d(x, W, b):
    return jnp.einsum('bcn,cd->bdn', x, W) + b[None, :, None]


def _mlp_groups(g, layers):
    for W, b in layers:
        g = jax.nn.relu(jnp.einsum('bskc,cd->bskd', g, W) + b)
    return g


def _ball_query(new_xyz, xyz, radius, nsample):
    d2 = (jnp.sum(new_xyz ** 2, -1)[:, :, None]
          + jnp.sum(xyz ** 2, -1)[:, None, :]
          - 2.0 * jnp.einsum('bsd,bnd->bsn', new_xyz, xyz))
    neg, idx = jax.lax.top_k(-d2, nsample)
    mask = (-neg) > radius * radius
    idx = jnp.where(mask, idx[:, :, :1], idx)
    return idx


def _gather_nd(points, idx):
    B = points.shape[0]
    return points[jnp.arange(B)[:, None, None], idx]


def _sa_layer(xyz, feats, npoint, radius, nsample, layers):
    inds = _fps_pallas(xyz, npoint)
    new_xyz = jnp.take_along_axis(xyz, inds[:, :, None], axis=1)
    idx = _ball_query(new_xyz, xyz, radius, nsample)
    grouped_xyz = _gather_nd(xyz, idx) - new_xyz[:, :, None, :]
    if feats is None:
        g = grouped_xyz
    else:
        g = _gather_nd(jnp.transpose(feats, (0, 2, 1)), idx)
    g = _mlp_groups(g, layers)
    newf = jnp.max(g, axis=2)
    return new_xyz, jnp.transpose(newf, (0, 2, 1)), inds


def _lfp(unknown_xyz, known_xyz, unknown_feats, known_feats, radius, nsample,
         layers, post):
    idx = _ball_query(unknown_xyz, known_xyz, radius, nsample)
    g = _gather_nd(jnp.transpose(known_feats, (0, 2, 1)), idx)
    g = _mlp_groups(g, layers)
    pooled = jnp.transpose(jnp.max(g, axis=2), (0, 2, 1))
    cat = jnp.concatenate([pooled, unknown_feats], axis=1)
    W, b = post
    return jax.nn.relu(_conv1d(cat, W, b))


def _fuse_text_seed(x, layers):
    (W1, b1), (W2, b2), (W3, b3) = layers
    x = jax.nn.relu(_conv1d(x, W1, b1))
    x = jax.nn.relu(_conv1d(x, W2, b2))
    return _conv1d(x, W3, b3)


def _vgen(seed_xyz, feat, layers):
    (W1, b1), (W2, b2), (W3, b3) = layers
    net = jax.nn.relu(_conv1d(feat, W1, b1))
    net = jax.nn.relu(_conv1d(net, W2, b2))
    net = _conv1d(net, W3, b3)
    offset = jnp.transpose(net[:, 0:3, :], (0, 2, 1))
    vote_xyz = seed_xyz + offset
    features = feat + net[:, 3:, :]
    return vote_xyz, features


def _vote_agg(vote_xyz, feats, layers, npoint=32, radius=0.3, nsample=16):
    inds = _fps_pallas(vote_xyz, npoint)
    new_xyz = jnp.take_along_axis(vote_xyz, inds[:, :, None], axis=1)
    idx = _ball_query(new_xyz, vote_xyz, radius, nsample)
    grouped_xyz = (_gather_nd(vote_xyz, idx) - new_xyz[:, :, None, :]) / radius
    gf = _gather_nd(jnp.transpose(feats, (0, 2, 1)), idx)
    g = jnp.concatenate([grouped_xyz, gf], axis=-1)
    g = _mlp_groups(g, layers)
    newf = jnp.transpose(jnp.max(g, axis=2), (0, 2, 1))
    return new_xyz, newf, inds


def _res_mlp(x, p):
    for (W1, b1), (W2, b2) in p['blocks']:
        y = jax.nn.relu(_conv1d(x, W1, b1))
        y = _conv1d(y, W2, b2)
        x = jax.nn.relu(x + y)
    W, b = p['final']
    return jax.nn.relu(_conv1d(x, W, b))


def kernel(input_point_cloud, lang_feat, pcd_heatmap, params):
    heat = jnp.clip(pcd_heatmap, 0.0, 1.0)
    B = input_point_cloud.shape[0]
    xyz0 = jnp.transpose(input_point_cloud[:, 0:3, :], (0, 2, 1))
    xyz1, f1, i1 = _sa_layer(xyz0, None, 2048, 0.1, 64, params['sa1'])
    xyz2, f2, i2 = _sa_layer(xyz1, f1, 1024, 0.2, 32, params['sa2'])
    key_xyz, pointwise_feat, i3 = _sa_layer(xyz2, f2, 512, 0.4, 16, params['sa3'])
    a = jnp.take_along_axis(i1, i2, axis=1)
    ind_512 = jnp.take_along_axis(a, i3, axis=1)
    w = jnp.power(heat + 1e-6, ALPHA)
    point_ind = _fps_pallas(xyz0, 256, w=w)
    seed_xyz = xyz0[jnp.arange(B)[:, None], point_ind]
    seed_feat = _lfp(seed_xyz, key_xyz, jnp.transpose(seed_xyz, (0, 2, 1)),
                     pointwise_feat, 0.4, 16, params['lfp1'], params['lfp1_post'])
    S = seed_feat.shape[2]
    cat_feat = jnp.concatenate(
        [seed_feat, jnp.repeat(lang_feat[:, :, None], S, axis=2)], axis=1)
    fuse_feat = _fuse_text_seed(cat_feat, params['fuse'])
    vote_xyz, features = _vgen(seed_xyz, fuse_feat, params['vgen'])
    fnorm = jnp.linalg.norm(features, axis=1, keepdims=True)
    features = features / fnorm
    xyz, features, fps_inds = _vote_agg(vote_xyz, features, params['va'])
    vote_inds = point_ind[jnp.arange(B)[:, None], fps_inds]
    feat = _res_mlp(features, params['res1'])
    partial_loc = _conv1d(feat, params['ploc'][0], params['ploc'][1])
    conf = jax.nn.sigmoid(_conv1d(feat, params['conf'][0], params['conf'][1]))
    partial_center = jnp.transpose(partial_loc[:, 0:3, :], (0, 2, 1)) + xyz
    intact_feat = _lfp(partial_center, seed_xyz, feat, fuse_feat, 0.4, 32,
                       params['lfp2'], params['lfp2_post'])
    intact_feat = _res_mlp(intact_feat, params['res2'])
    intact_loc = _conv1d(intact_feat, params['iloc'][0], params['iloc'][1])
    vote_heatmap = heat[jnp.arange(B)[:, None], vote_inds]
    pseudo_seed_feat = _lfp(key_xyz, key_xyz, jnp.transpose(key_xyz, (0, 2, 1)),
                            pointwise_feat, 0.4, 16, params['lfp1'],
                            params['lfp1_post'])
    S2 = pseudo_seed_feat.shape[2]
    pseudo_cat = jnp.concatenate(
        [pseudo_seed_feat, jnp.repeat(lang_feat[:, :, None], S2, axis=2)], axis=1)
    pseudo_fuse = _fuse_text_seed(pseudo_cat, params['fuse'])
    pseudo_vote_xyz, _ = _vgen(key_xyz, pseudo_fuse, params['vgen'])
    return {'cluster_loc': jnp.transpose(xyz, (0, 2, 1)),
            'vote_heatmap': vote_heatmap,
            'vote_loc': jnp.transpose(vote_xyz, (0, 2, 1)),
            'seed_loc': jnp.transpose(seed_xyz, (0, 2, 1)),
            'seed_ind': point_ind,
            'vote_inds': vote_inds,
            'pred_partial_loc': partial_loc,
            'pred_intact_loc': intact_loc,
            'pred_conf': conf,
            'pcd_heatmap': heat,
            'pseudo_seed_ind': ind_512,
            'pseudo_vote_loc': jnp.transpose(pseudo_vote_xyz, (0, 2, 1))}


# speedup vs baseline: 2.9065x; 2.9065x over previous
import functools

import jax
import jax.numpy as jnp
from jax.experimental import pallas as pl
from jax.experimental.pallas import tpu as pltpu
from jax.experimental.pallas import tpu_sc as plsc

ALPHA = 0.5
T = 1.0


def _argmax2d(v, fiota):
    m = jnp.max(v)
    return jnp.min(jnp.where(v == m, fiota, 2147483647))


def _fps_kern(x_ref, y_ref, z_ref, *rest, npoint, weighted, out_rows):
    if weighted:
        w_ref, out_ref = rest
    else:
        (out_ref,) = rest
    x = x_ref[0]
    y = y_ref[0]
    z = z_ref[0]
    R, C = x.shape
    riota = jax.lax.broadcasted_iota(jnp.int32, (R, C), 0)
    ciota = jax.lax.broadcasted_iota(jnp.int32, (R, C), 1)
    fiota = riota * C + ciota
    oriota = jax.lax.broadcasted_iota(jnp.int32, (out_rows, 128), 0)
    ociota = jax.lax.broadcasted_iota(jnp.int32, (out_rows, 128), 1)
    oiota = oriota * 128 + ociota
    if weighted:
        w = w_ref[0]
        far0 = _argmax2d(w, fiota)
    else:
        far0 = jnp.int32(0)
    dists0 = jnp.full((R, C), 1e10, jnp.float32)
    inds0 = jnp.zeros((out_rows, 128), jnp.int32)

    def step(i, carry):
        dists, inds, far = carry
        sel = fiota == far
        ninf = jnp.float32(-jnp.inf)
        cx = jnp.max(jnp.where(sel, x, ninf))
        cy = jnp.max(jnp.where(sel, y, ninf))
        cz = jnp.max(jnp.where(sel, z, ninf))
        dx = x - cx
        dy = y - cy
        dz = z - cz
        d = dx * dx + dy * dy + dz * dz
        dists = jnp.minimum(dists, d)
        inds = jnp.where(oiota == i, far, inds)
        if weighted:
            score = jnp.maximum(dists, 0.0) * w
        else:
            score = dists
        nxt = _argmax2d(score, fiota)
        return dists, inds, nxt

    _, inds, _ = jax.lax.fori_loop(0, npoint, step, (dists0, inds0, far0))
    out_ref[0] = inds


def _fps_pallas(xyz, npoint, w=None):
    B, N, _ = xyz.shape
    C = 128
    R = N // C
    assert R * C == N
    out_rows = max(1, (npoint + 127) // 128)
    xr = xyz[:, :, 0].reshape(B, R, C)
    yr = xyz[:, :, 1].reshape(B, R, C)
    zr = xyz[:, :, 2].reshape(B, R, C)
    weighted = w is not None
    args = [xr, yr, zr]
    if weighted:
        args.append(w.reshape(B, R, C))
    spec = pl.BlockSpec((1, R, C), lambda b: (b, 0, 0))
    in_specs = [spec] * len(args)
    out = pl.pallas_call(
        functools.partial(_fps_kern, npoint=npoint, weighted=weighted,
                          out_rows=out_rows),
        grid=(B,),
        in_specs=in_specs,
        out_specs=pl.BlockSpec((1, out_rows, 128), lambda b: (b, 0, 0)),
        out_shape=jax.ShapeDtypeStruct((B, out_rows, 128), jnp.int32),
    )(*args)
    return out.reshape(B, out_rows * 128)[:, :npoint]


def _bq_sc_body(cx_h, cy_h, cz_h, x_h, y_h, z_h, out_h,
                xs, ys, zs, cdd, cdi, outv, cxs, cys, czs,
                *, B, S, N, K, KS, WPB, RPW, r2, NC):
    c = jax.lax.axis_index("c")
    s = jax.lax.axis_index("s")
    wid = s * NC + c
    b = wid // WPB
    ww = wid % WPB
    pltpu.sync_copy(x_h.at[b], xs)
    pltpu.sync_copy(y_h.at[b], ys)
    pltpu.sync_copy(z_h.at[b], zs)
    base = ww * RPW
    pltpu.sync_copy(cx_h.at[b, pl.ds(base, RPW)], cxs)
    pltpu.sync_copy(cy_h.at[b, pl.ds(base, RPW)], cys)
    pltpu.sync_copy(cz_h.at[b, pl.ds(base, RPW)], czs)
    iota = jax.lax.iota(jnp.int32, 16)
    NCH = N // 16

    def row_fn(r, _):
        c0 = (r // 16) * 16
        lsel = iota == (r - c0)
        lself = lsel.astype(jnp.float32)
        cxv = jnp.full((16,), jnp.sum(cxs[pl.ds(c0, 16)] * lself), jnp.float32)
        cyv = jnp.full((16,), jnp.sum(cys[pl.ds(c0, 16)] * lself), jnp.float32)
        czv = jnp.full((16,), jnp.sum(czs[pl.ds(c0, 16)] * lself), jnp.float32)

        def scan_ch(j, car):
            cnt, minv, mini = car
            sl = pl.ds(j * 16, 16)
            dx = xs[sl] - cxv
            dy = ys[sl] - cyv
            dz = zs[sl] - czv
            d2 = dx * dx + dy * dy + dz * dz
            d2c = jnp.maximum(d2, jnp.float32(0.0))
            iv = iota + j * 16
            m = d2 <= jnp.float32(r2)
            plsc.store_compressed(cdd.at[pl.ds(cnt, 16)], d2c, mask=m)
            plsc.store_compressed(cdi.at[pl.ds(cnt, 16)], iv, mask=m)
            cnt = cnt + jnp.sum(m.astype(jnp.int32))
            mm = d2c < minv
            minv = jnp.where(mm, d2c, minv)
            mini = jnp.where(mm, iv, mini)
            return cnt, minv, mini

        cnt, minv, mini = jax.lax.fori_loop(
            0, NCH, scan_ch,
            (jnp.int32(0), jnp.full((16,), jnp.inf, jnp.float32),
             jnp.zeros((16,), jnp.int32)))
        mn = jnp.min(minv)
        nearest = jnp.min(jnp.where(minv == mn, mini, 2147483647))
        nchc = (cnt + 15) // 16

        def bit_fn(i, acc):
            t = acc | (1 << (30 - i))

            def cnt_ch(j, c2):
                bits = plsc.bitcast(cdd[pl.ds(j * 16, 16)], jnp.int32)
                le = (bits < t) & ((iota + j * 16) < cnt)
                return c2 + jnp.sum(le.astype(jnp.int32))

            cless = jax.lax.fori_loop(0, nchc, cnt_ch, jnp.int32(0))
            return jnp.where(cless < K, t, acc)

        tau = jax.lax.fori_loop(0, 31, bit_fn, jnp.int32(0))
        nearv = jnp.full((16,), nearest, jnp.int32)
        for cslot in range(KS // 16):
            outv[cslot * 16:(cslot + 1) * 16] = nearv

        def emit_ch(j, pos):
            sl = pl.ds(j * 16, 16)
            bits = plsc.bitcast(cdd[sl], jnp.int32)
            le = (bits <= tau) & ((iota + j * 16) < cnt)
            pref = plsc.cumsum(le.astype(jnp.int32))
            keep = le & ((pref + pos) <= K)
            plsc.store_compressed(outv.at[pl.ds(pos, 16)], cdi[sl], mask=keep)
            return pos + jnp.sum(keep.astype(jnp.int32))

        jax.lax.fori_loop(0, nchc, emit_ch, jnp.int32(0))
        row = b * S + base + r
        pltpu.sync_copy(outv.at[pl.ds(0, K)], out_h.at[pl.ds(row * K, K)])
        return 0

    jax.lax.fori_loop(0, RPW, row_fn, 0)


def _ball_query_sc(new_xyz, xyz, radius, nsample):
    B, S, _ = new_xyz.shape
    N = xyz.shape[1]
    info = plsc.get_sparse_core_info()
    NC, NS = info.num_cores, info.num_subcores
    NW = NC * NS
    WPB = NW // B
    RPW = S // WPB
    K = nsample
    KS = K + 16
    r2 = float(radius) * float(radius)
    body = functools.partial(_bq_sc_body, B=B, S=S, N=N, K=K, KS=KS,
                             WPB=WPB, RPW=RPW, r2=r2, NC=NC)
    mesh = plsc.VectorSubcoreMesh(core_axis_name="c", subcore_axis_name="s")
    fn = pl.kernel(
        body, mesh=mesh,
        compiler_params=pltpu.CompilerParams(needs_layout_passes=False),
        out_type=jax.ShapeDtypeStruct((B * S * K,), jnp.int32),
        scratch_types=[
            pltpu.VMEM((N,), jnp.float32),
            pltpu.VMEM((N,), jnp.float32),
            pltpu.VMEM((N,), jnp.float32),
            pltpu.VMEM((N + 16,), jnp.float32),
            pltpu.VMEM((N + 16,), jnp.int32),
            pltpu.VMEM((KS,), jnp.int32),
            pltpu.VMEM((RPW,), jnp.float32),
            pltpu.VMEM((RPW,), jnp.float32),
            pltpu.VMEM((RPW,), jnp.float32),
        ])
    out = fn(new_xyz[..., 0].reshape(B, S), new_xyz[..., 1].reshape(B, S),
             new_xyz[..., 2].reshape(B, S), xyz[..., 0].reshape(B, N),
             xyz[..., 1].reshape(B, N), xyz[..., 2].reshape(B, N))
    return out.reshape(B, S, K)


def _conv1d(x, W, b):
    return jnp.einsum('bcn,cd->bdn', x, W) + b[None, :, None]


def _mlp_groups(g, layers):
    for W, b in layers:
        g = jax.nn.relu(jnp.einsum('bskc,cd->bskd', g, W) + b)
    return g


def _ball_query(new_xyz, xyz, radius, nsample):
    d2 = (jnp.sum(new_xyz ** 2, -1)[:, :, None]
          + jnp.sum(xyz ** 2, -1)[:, None, :]
          - 2.0 * jnp.einsum('bsd,bnd->bsn', new_xyz, xyz))
    neg, idx = jax.lax.top_k(-d2, nsample)
    mask = (-neg) > radius * radius
    idx = jnp.where(mask, idx[:, :, :1], idx)
    return idx


def _gather_nd(points, idx):
    B = points.shape[0]
    return points[jnp.arange(B)[:, None, None], idx]


def _sa_layer(xyz, feats, npoint, radius, nsample, layers):
    inds = _fps_pallas(xyz, npoint)
    new_xyz = jnp.take_along_axis(xyz, inds[:, :, None], axis=1)
    idx = _ball_query_sc(new_xyz, xyz, radius, nsample)
    grouped_xyz = _gather_nd(xyz, idx) - new_xyz[:, :, None, :]
    if feats is None:
        g = grouped_xyz
    else:
        g = _gather_nd(jnp.transpose(feats, (0, 2, 1)), idx)
    g = _mlp_groups(g, layers)
    newf = jnp.max(g, axis=2)
    return new_xyz, jnp.transpose(newf, (0, 2, 1)), inds


def _lfp(unknown_xyz, known_xyz, unknown_feats, known_feats, radius, nsample,
         layers, post):
    idx = _ball_query(unknown_xyz, known_xyz, radius, nsample)
    g = _gather_nd(jnp.transpose(known_feats, (0, 2, 1)), idx)
    g = _mlp_groups(g, layers)
    pooled = jnp.transpose(jnp.max(g, axis=2), (0, 2, 1))
    cat = jnp.concatenate([pooled, unknown_feats], axis=1)
    W, b = post
    return jax.nn.relu(_conv1d(cat, W, b))


def _fuse_text_seed(x, layers):
    (W1, b1), (W2, b2), (W3, b3) = layers
    x = jax.nn.relu(_conv1d(x, W1, b1))
    x = jax.nn.relu(_conv1d(x, W2, b2))
    return _conv1d(x, W3, b3)


def _vgen(seed_xyz, feat, layers):
    (W1, b1), (W2, b2), (W3, b3) = layers
    net = jax.nn.relu(_conv1d(feat, W1, b1))
    net = jax.nn.relu(_conv1d(net, W2, b2))
    net = _conv1d(net, W3, b3)
    offset = jnp.transpose(net[:, 0:3, :], (0, 2, 1))
    vote_xyz = seed_xyz + offset
    features = feat + net[:, 3:, :]
    return vote_xyz, features


def _vote_agg(vote_xyz, feats, layers, npoint=32, radius=0.3, nsample=16):
    inds = _fps_pallas(vote_xyz, npoint)
    new_xyz = jnp.take_along_axis(vote_xyz, inds[:, :, None], axis=1)
    idx = _ball_query(new_xyz, vote_xyz, radius, nsample)
    grouped_xyz = (_gather_nd(vote_xyz, idx) - new_xyz[:, :, None, :]) / radius
    gf = _gather_nd(jnp.transpose(feats, (0, 2, 1)), idx)
    g = jnp.concatenate([grouped_xyz, gf], axis=-1)
    g = _mlp_groups(g, layers)
    newf = jnp.transpose(jnp.max(g, axis=2), (0, 2, 1))
    return new_xyz, newf, inds


def _res_mlp(x, p):
    for (W1, b1), (W2, b2) in p['blocks']:
        y = jax.nn.relu(_conv1d(x, W1, b1))
        y = _conv1d(y, W2, b2)
        x = jax.nn.relu(x + y)
    W, b = p['final']
    return jax.nn.relu(_conv1d(x, W, b))


def kernel(input_point_cloud, lang_feat, pcd_heatmap, params):
    heat = jnp.clip(pcd_heatmap, 0.0, 1.0)
    B = input_point_cloud.shape[0]
    xyz0 = jnp.transpose(input_point_cloud[:, 0:3, :], (0, 2, 1))
    xyz1, f1, i1 = _sa_layer(xyz0, None, 2048, 0.1, 64, params['sa1'])
    xyz2, f2, i2 = _sa_layer(xyz1, f1, 1024, 0.2, 32, params['sa2'])
    key_xyz, pointwise_feat, i3 = _sa_layer(xyz2, f2, 512, 0.4, 16, params['sa3'])
    a = jnp.take_along_axis(i1, i2, axis=1)
    ind_512 = jnp.take_along_axis(a, i3, axis=1)
    w = jnp.power(heat + 1e-6, ALPHA)
    point_ind = _fps_pallas(xyz0, 256, w=w)
    seed_xyz = xyz0[jnp.arange(B)[:, None], point_ind]
    seed_feat = _lfp(seed_xyz, key_xyz, jnp.transpose(seed_xyz, (0, 2, 1)),
                     pointwise_feat, 0.4, 16, params['lfp1'], params['lfp1_post'])
    S = seed_feat.shape[2]
    cat_feat = jnp.concatenate(
        [seed_feat, jnp.repeat(lang_feat[:, :, None], S, axis=2)], axis=1)
    fuse_feat = _fuse_text_seed(cat_feat, params['fuse'])
    vote_xyz, features = _vgen(seed_xyz, fuse_feat, params['vgen'])
    fnorm = jnp.linalg.norm(features, axis=1, keepdims=True)
    features = features / fnorm
    xyz, features, fps_inds = _vote_agg(vote_xyz, features, params['va'])
    vote_inds = point_ind[jnp.arange(B)[:, None], fps_inds]
    feat = _res_mlp(features, params['res1'])
    partial_loc = _conv1d(feat, params['ploc'][0], params['ploc'][1])
    conf = jax.nn.sigmoid(_conv1d(feat, params['conf'][0], params['conf'][1]))
    partial_center = jnp.transpose(partial_loc[:, 0:3, :], (0, 2, 1)) + xyz
    intact_feat = _lfp(partial_center, seed_xyz, feat, fuse_feat, 0.4, 32,
                       params['lfp2'], params['lfp2_post'])
    intact_feat = _res_mlp(intact_feat, params['res2'])
    intact_loc = _conv1d(intact_feat, params['iloc'][0], params['iloc'][1])
    vote_heatmap = heat[jnp.arange(B)[:, None], vote_inds]
    pseudo_seed_feat = _lfp(key_xyz, key_xyz, jnp.transpose(key_xyz, (0, 2, 1)),
                            pointwise_feat, 0.4, 16, params['lfp1'],
                            params['lfp1_post'])
    S2 = pseudo_seed_feat.shape[2]
    pseudo_cat = jnp.concatenate(
        [pseudo_seed_feat, jnp.repeat(lang_feat[:, :, None], S2, axis=2)], axis=1)
    pseudo_fuse = _fuse_text_seed(pseudo_cat, params['fuse'])
    pseudo_vote_xyz, _ = _vgen(key_xyz, pseudo_fuse, params['vgen'])
    return {'cluster_loc': jnp.transpose(xyz, (0, 2, 1)),
            'vote_heatmap': vote_heatmap,
            'vote_loc': jnp.transpose(vote_xyz, (0, 2, 1)),
            'seed_loc': jnp.transpose(seed_xyz, (0, 2, 1)),
            'seed_ind': point_ind,
            'vote_inds': vote_inds,
            'pred_partial_loc': partial_loc,
            'pred_intact_loc': intact_loc,
            'pred_conf': conf,
            'pcd_heatmap': heat,
            'pseudo_seed_ind': ind_512,
            'pseudo_vote_loc': jnp.transpose(pseudo_vote_xyz, (0, 2, 1))}


# speedup vs baseline: 2.9088x; 1.0008x over previous
import functools

import jax
import jax.numpy as jnp
from jax.experimental import pallas as pl
from jax.experimental.pallas import tpu as pltpu
from jax.experimental.pallas import tpu_sc as plsc

ALPHA = 0.5
T = 1.0


def _argmax2d(v, fiota):
    m = jnp.max(v)
    return jnp.min(jnp.where(v == m, fiota, 2147483647))


def _fps_kern(x_ref, y_ref, z_ref, *rest, npoint, weighted, out_rows):
    if weighted:
        w_ref, out_ref = rest
    else:
        (out_ref,) = rest
    x = x_ref[0]
    y = y_ref[0]
    z = z_ref[0]
    R, C = x.shape
    riota = jax.lax.broadcasted_iota(jnp.int32, (R, C), 0)
    ciota = jax.lax.broadcasted_iota(jnp.int32, (R, C), 1)
    fiota = riota * C + ciota
    oriota = jax.lax.broadcasted_iota(jnp.int32, (out_rows, 128), 0)
    ociota = jax.lax.broadcasted_iota(jnp.int32, (out_rows, 128), 1)
    oiota = oriota * 128 + ociota
    if weighted:
        w = w_ref[0]
        far0 = _argmax2d(w, fiota)
    else:
        far0 = jnp.int32(0)
    dists0 = jnp.full((R, C), 1e10, jnp.float32)
    inds0 = jnp.zeros((out_rows, 128), jnp.int32)

    def step(i, carry):
        dists, inds, far = carry
        sel = fiota == far
        ninf = jnp.float32(-jnp.inf)
        cx = jnp.max(jnp.where(sel, x, ninf))
        cy = jnp.max(jnp.where(sel, y, ninf))
        cz = jnp.max(jnp.where(sel, z, ninf))
        dx = x - cx
        dy = y - cy
        dz = z - cz
        d = dx * dx + dy * dy + dz * dz
        dists = jnp.minimum(dists, d)
        inds = jnp.where(oiota == i, far, inds)
        if weighted:
            score = jnp.maximum(dists, 0.0) * w
        else:
            score = dists
        nxt = _argmax2d(score, fiota)
        return dists, inds, nxt

    _, inds, _ = jax.lax.fori_loop(0, npoint, step, (dists0, inds0, far0))
    out_ref[0] = inds


def _fps_pallas(xyz, npoint, w=None):
    B, N, _ = xyz.shape
    C = 128
    R = N // C
    assert R * C == N
    out_rows = max(1, (npoint + 127) // 128)
    xr = xyz[:, :, 0].reshape(B, R, C)
    yr = xyz[:, :, 1].reshape(B, R, C)
    zr = xyz[:, :, 2].reshape(B, R, C)
    weighted = w is not None
    args = [xr, yr, zr]
    if weighted:
        args.append(w.reshape(B, R, C))
    spec = pl.BlockSpec((1, R, C), lambda b: (b, 0, 0))
    in_specs = [spec] * len(args)
    out = pl.pallas_call(
        functools.partial(_fps_kern, npoint=npoint, weighted=weighted,
                          out_rows=out_rows),
        grid=(B,),
        compiler_params=pltpu.CompilerParams(
            dimension_semantics=("parallel",)),
        in_specs=in_specs,
        out_specs=pl.BlockSpec((1, out_rows, 128), lambda b: (b, 0, 0)),
        out_shape=jax.ShapeDtypeStruct((B, out_rows, 128), jnp.int32),
    )(*args)
    return out.reshape(B, out_rows * 128)[:, :npoint]


def _bq_sc_body(cx_h, cy_h, cz_h, x_h, y_h, z_h, out_h,
                xs, ys, zs, cdd, cdi, outv, cxs, cys, czs,
                *, B, S, N, K, KS, WPB, RPW, r2, NC):
    c = jax.lax.axis_index("c")
    s = jax.lax.axis_index("s")
    wid = s * NC + c
    b = wid // WPB
    ww = wid % WPB
    pltpu.sync_copy(x_h.at[b], xs)
    pltpu.sync_copy(y_h.at[b], ys)
    pltpu.sync_copy(z_h.at[b], zs)
    base = ww * RPW
    pltpu.sync_copy(cx_h.at[b, pl.ds(base, RPW)], cxs.at[pl.ds(0, RPW)])
    pltpu.sync_copy(cy_h.at[b, pl.ds(base, RPW)], cys.at[pl.ds(0, RPW)])
    pltpu.sync_copy(cz_h.at[b, pl.ds(base, RPW)], czs.at[pl.ds(0, RPW)])
    iota = jax.lax.iota(jnp.int32, 16)
    NCH = N // 16

    def row_fn(r, _):
        c0 = (r // 16) * 16
        lsel = iota == (r - c0)
        lself = lsel.astype(jnp.float32)
        cxv = jnp.full((16,), jnp.sum(cxs[pl.ds(c0, 16)] * lself), jnp.float32)
        cyv = jnp.full((16,), jnp.sum(cys[pl.ds(c0, 16)] * lself), jnp.float32)
        czv = jnp.full((16,), jnp.sum(czs[pl.ds(c0, 16)] * lself), jnp.float32)

        def scan_ch(j, car):
            cnt, minv, mini = car
            sl = pl.ds(j * 16, 16)
            dx = xs[sl] - cxv
            dy = ys[sl] - cyv
            dz = zs[sl] - czv
            d2 = dx * dx + dy * dy + dz * dz
            d2c = jnp.maximum(d2, jnp.float32(0.0))
            iv = iota + j * 16
            m = d2 <= jnp.float32(r2)
            plsc.store_compressed(cdd.at[pl.ds(cnt, 16)], d2c, mask=m)
            plsc.store_compressed(cdi.at[pl.ds(cnt, 16)], iv, mask=m)
            cnt = cnt + jnp.sum(m.astype(jnp.int32))
            mm = d2c < minv
            minv = jnp.where(mm, d2c, minv)
            mini = jnp.where(mm, iv, mini)
            return cnt, minv, mini

        cnt, minv, mini = jax.lax.fori_loop(
            0, NCH, scan_ch,
            (jnp.int32(0), jnp.full((16,), jnp.inf, jnp.float32),
             jnp.zeros((16,), jnp.int32)))
        mn = jnp.min(minv)
        nearest = jnp.min(jnp.where(minv == mn, mini, 2147483647))
        nchc = (cnt + 15) // 16

        def bit_fn(i, acc):
            t = acc | (1 << (30 - i))

            def cnt_ch(j, c2):
                bits = plsc.bitcast(cdd[pl.ds(j * 16, 16)], jnp.int32)
                le = (bits < t) & ((iota + j * 16) < cnt)
                return c2 + jnp.sum(le.astype(jnp.int32))

            cless = jax.lax.fori_loop(0, nchc, cnt_ch, jnp.int32(0))
            return jnp.where(cless < K, t, acc)

        tau = jax.lax.fori_loop(0, 31, bit_fn, jnp.int32(0))
        nearv = jnp.full((16,), nearest, jnp.int32)
        for cslot in range(KS // 16):
            outv[cslot * 16:(cslot + 1) * 16] = nearv

        def emit_ch(j, pos):
            sl = pl.ds(j * 16, 16)
            bits = plsc.bitcast(cdd[sl], jnp.int32)
            le = (bits <= tau) & ((iota + j * 16) < cnt)
            pref = plsc.cumsum(le.astype(jnp.int32))
            keep = le & ((pref + pos) <= K)
            plsc.store_compressed(outv.at[pl.ds(pos, 16)], cdi[sl], mask=keep)
            return pos + jnp.sum(keep.astype(jnp.int32))

        jax.lax.fori_loop(0, nchc, emit_ch, jnp.int32(0))
        row = b * S + base + r
        pltpu.sync_copy(outv.at[pl.ds(0, K)], out_h.at[pl.ds(row * K, K)])
        return 0

    jax.lax.fori_loop(0, RPW, row_fn, 0)


def _ball_query_sc(new_xyz, xyz, radius, nsample):
    B, S, _ = new_xyz.shape
    N = xyz.shape[1]
    info = plsc.get_sparse_core_info()
    NC, NS = info.num_cores, info.num_subcores
    NW = NC * NS
    WPB = NW // B
    RPW = S // WPB
    K = nsample
    KS = K + 16
    r2 = float(radius) * float(radius)
    body = functools.partial(_bq_sc_body, B=B, S=S, N=N, K=K, KS=KS,
                             WPB=WPB, RPW=RPW, r2=r2, NC=NC)
    mesh = plsc.VectorSubcoreMesh(core_axis_name="c", subcore_axis_name="s")
    fn = pl.kernel(
        body, mesh=mesh,
        compiler_params=pltpu.CompilerParams(needs_layout_passes=False),
        out_type=jax.ShapeDtypeStruct((B * S * K,), jnp.int32),
        scratch_types=[
            pltpu.VMEM((N,), jnp.float32),
            pltpu.VMEM((N,), jnp.float32),
            pltpu.VMEM((N,), jnp.float32),
            pltpu.VMEM((N + 16,), jnp.float32),
            pltpu.VMEM((N + 16,), jnp.int32),
            pltpu.VMEM((KS,), jnp.int32),
            pltpu.VMEM((max(RPW, 16),), jnp.float32),
            pltpu.VMEM((max(RPW, 16),), jnp.float32),
            pltpu.VMEM((max(RPW, 16),), jnp.float32),
        ])
    out = fn(new_xyz[..., 0].reshape(B, S), new_xyz[..., 1].reshape(B, S),
             new_xyz[..., 2].reshape(B, S), xyz[..., 0].reshape(B, N),
             xyz[..., 1].reshape(B, N), xyz[..., 2].reshape(B, N))
    return out.reshape(B, S, K)


def _conv1d(x, W, b):
    return jnp.einsum('bcn,cd->bdn', x, W) + b[None, :, None]


def _mlp_groups(g, layers):
    for W, b in layers:
        g = jax.nn.relu(jnp.einsum('bskc,cd->bskd', g, W) + b)
    return g


def _ball_query(new_xyz, xyz, radius, nsample):
    d2 = (jnp.sum(new_xyz ** 2, -1)[:, :, None]
          + jnp.sum(xyz ** 2, -1)[:, None, :]
          - 2.0 * jnp.einsum('bsd,bnd->bsn', new_xyz, xyz))
    neg, idx = jax.lax.top_k(-d2, nsample)
    mask = (-neg) > radius * radius
    idx = jnp.where(mask, idx[:, :, :1], idx)
    return idx


def _gather_nd(points, idx):
    B = points.shape[0]
    return points[jnp.arange(B)[:, None, None], idx]


def _sa_layer(xyz, feats, npoint, radius, nsample, layers):
    inds = _fps_pallas(xyz, npoint)
    new_xyz = jnp.take_along_axis(xyz, inds[:, :, None], axis=1)
    idx = _ball_query_sc(new_xyz, xyz, radius, nsample)
    grouped_xyz = _gather_nd(xyz, idx) - new_xyz[:, :, None, :]
    if feats is None:
        g = grouped_xyz
    else:
        g = _gather_nd(jnp.transpose(feats, (0, 2, 1)), idx)
    g = _mlp_groups(g, layers)
    newf = jnp.max(g, axis=2)
    return new_xyz, jnp.transpose(newf, (0, 2, 1)), inds


def _lfp(unknown_xyz, known_xyz, unknown_feats, known_feats, radius, nsample,
         layers, post):
    idx = _ball_query(unknown_xyz, known_xyz, radius, nsample)
    g = _gather_nd(jnp.transpose(known_feats, (0, 2, 1)), idx)
    g = _mlp_groups(g, layers)
    pooled = jnp.transpose(jnp.max(g, axis=2), (0, 2, 1))
    cat = jnp.concatenate([pooled, unknown_feats], axis=1)
    W, b = post
    return jax.nn.relu(_conv1d(cat, W, b))


def _fuse_text_seed(x, layers):
    (W1, b1), (W2, b2), (W3, b3) = layers
    x = jax.nn.relu(_conv1d(x, W1, b1))
    x = jax.nn.relu(_conv1d(x, W2, b2))
    return _conv1d(x, W3, b3)


def _vgen(seed_xyz, feat, layers):
    (W1, b1), (W2, b2), (W3, b3) = layers
    net = jax.nn.relu(_conv1d(feat, W1, b1))
    net = jax.nn.relu(_conv1d(net, W2, b2))
    net = _conv1d(net, W3, b3)
    offset = jnp.transpose(net[:, 0:3, :], (0, 2, 1))
    vote_xyz = seed_xyz + offset
    features = feat + net[:, 3:, :]
    return vote_xyz, features


def _vote_agg(vote_xyz, feats, layers, npoint=32, radius=0.3, nsample=16):
    inds = _fps_pallas(vote_xyz, npoint)
    new_xyz = jnp.take_along_axis(vote_xyz, inds[:, :, None], axis=1)
    idx = _ball_query(new_xyz, vote_xyz, radius, nsample)
    grouped_xyz = (_gather_nd(vote_xyz, idx) - new_xyz[:, :, None, :]) / radius
    gf = _gather_nd(jnp.transpose(feats, (0, 2, 1)), idx)
    g = jnp.concatenate([grouped_xyz, gf], axis=-1)
    g = _mlp_groups(g, layers)
    newf = jnp.transpose(jnp.max(g, axis=2), (0, 2, 1))
    return new_xyz, newf, inds


def _res_mlp(x, p):
    for (W1, b1), (W2, b2) in p['blocks']:
        y = jax.nn.relu(_conv1d(x, W1, b1))
        y = _conv1d(y, W2, b2)
        x = jax.nn.relu(x + y)
    W, b = p['final']
    return jax.nn.relu(_conv1d(x, W, b))


def kernel(input_point_cloud, lang_feat, pcd_heatmap, params):
    heat = jnp.clip(pcd_heatmap, 0.0, 1.0)
    B = input_point_cloud.shape[0]
    xyz0 = jnp.transpose(input_point_cloud[:, 0:3, :], (0, 2, 1))
    xyz1, f1, i1 = _sa_layer(xyz0, None, 2048, 0.1, 64, params['sa1'])
    xyz2, f2, i2 = _sa_layer(xyz1, f1, 1024, 0.2, 32, params['sa2'])
    key_xyz, pointwise_feat, i3 = _sa_layer(xyz2, f2, 512, 0.4, 16, params['sa3'])
    a = jnp.take_along_axis(i1, i2, axis=1)
    ind_512 = jnp.take_along_axis(a, i3, axis=1)
    w = jnp.power(heat + 1e-6, ALPHA)
    point_ind = _fps_pallas(xyz0, 256, w=w)
    seed_xyz = xyz0[jnp.arange(B)[:, None], point_ind]
    seed_feat = _lfp(seed_xyz, key_xyz, jnp.transpose(seed_xyz, (0, 2, 1)),
                     pointwise_feat, 0.4, 16, params['lfp1'], params['lfp1_post'])
    S = seed_feat.shape[2]
    cat_feat = jnp.concatenate(
        [seed_feat, jnp.repeat(lang_feat[:, :, None], S, axis=2)], axis=1)
    fuse_feat = _fuse_text_seed(cat_feat, params['fuse'])
    vote_xyz, features = _vgen(seed_xyz, fuse_feat, params['vgen'])
    fnorm = jnp.linalg.norm(features, axis=1, keepdims=True)
    features = features / fnorm
    xyz, features, fps_inds = _vote_agg(vote_xyz, features, params['va'])
    vote_inds = point_ind[jnp.arange(B)[:, None], fps_inds]
    feat = _res_mlp(features, params['res1'])
    partial_loc = _conv1d(feat, params['ploc'][0], params['ploc'][1])
    conf = jax.nn.sigmoid(_conv1d(feat, params['conf'][0], params['conf'][1]))
    partial_center = jnp.transpose(partial_loc[:, 0:3, :], (0, 2, 1)) + xyz
    intact_feat = _lfp(partial_center, seed_xyz, feat, fuse_feat, 0.4, 32,
                       params['lfp2'], params['lfp2_post'])
    intact_feat = _res_mlp(intact_feat, params['res2'])
    intact_loc = _conv1d(intact_feat, params['iloc'][0], params['iloc'][1])
    vote_heatmap = heat[jnp.arange(B)[:, None], vote_inds]
    pseudo_seed_feat = _lfp(key_xyz, key_xyz, jnp.transpose(key_xyz, (0, 2, 1)),
                            pointwise_feat, 0.4, 16, params['lfp1'],
                            params['lfp1_post'])
    S2 = pseudo_seed_feat.shape[2]
    pseudo_cat = jnp.concatenate(
        [pseudo_seed_feat, jnp.repeat(lang_feat[:, :, None], S2, axis=2)], axis=1)
    pseudo_fuse = _fuse_text_seed(pseudo_cat, params['fuse'])
    pseudo_vote_xyz, _ = _vgen(key_xyz, pseudo_fuse, params['vgen'])
    return {'cluster_loc': jnp.transpose(xyz, (0, 2, 1)),
            'vote_heatmap': vote_heatmap,
            'vote_loc': jnp.transpose(vote_xyz, (0, 2, 1)),
            'seed_loc': jnp.transpose(seed_xyz, (0, 2, 1)),
            'seed_ind': point_ind,
            'vote_inds': vote_inds,
            'pred_partial_loc': partial_loc,
            'pred_intact_loc': intact_loc,
            'pred_conf': conf,
            'pcd_heatmap': heat,
            'pseudo_seed_ind': ind_512,
            'pseudo_vote_loc': jnp.transpose(pseudo_vote_xyz, (0, 2, 1))}
